# baseline (speedup 1.0000x reference)
import functools

import jax
import jax.numpy as jnp
from jax import lax
from jax.experimental import pallas as pl
from jax.experimental.pallas import tpu as pltpu
from jax.experimental.pallas import tpu_sc as plsc

NUM_BINS = 65536
N = 16777216
L = 16
NC = 2
NS = 16
NW = NC * NS
PER_W = N // NW
CHUNK = 16384
NCHUNK = PER_W // CHUNK
VPC = CHUNK // L
UNROLL = 16
NBUF = 3
ZUNROLL = 16


def _sc_bincount(x):
    mesh = plsc.VectorSubcoreMesh(core_axis_name="c", subcore_axis_name="s")

    @functools.partial(
        pl.kernel,
        mesh=mesh,
        compiler_params=pltpu.CompilerParams(needs_layout_passes=False),
        out_type=jax.ShapeDtypeStruct((NW, NUM_BINS), jnp.int32),
        scratch_types=[
            pltpu.VMEM((NUM_BINS,), jnp.int32),
        ]
        + [pltpu.VMEM((CHUNK,), jnp.int32)] * NBUF
        + [pltpu.SemaphoreType.DMA] * NBUF,
    )
    def body(x_hbm, out_hbm, hist, *rest):
        bufs = rest[:NBUF]
        sems = rest[NBUF:]
        c = lax.axis_index("c")
        s = lax.axis_index("s")
        wid = s * NC + c
        base = wid * PER_W

        def start(g, b):
            return pltpu.async_copy(
                x_hbm.at[pl.ds(base + g * CHUNK, CHUNK)], bufs[b], sems[b]
            )

        for b in range(NBUF):
            start(b, b)

        zeros16 = jnp.zeros((L,), jnp.int32)

        def zero_body(i, carry):
            for u in range(ZUNROLL):
                hist[pl.ds((i * ZUNROLL + u) * L, L)] = zeros16
            return carry

        lax.fori_loop(0, NUM_BINS // (L * ZUNROLL), zero_body, 0)

        ones16 = jnp.full((L,), 1, jnp.int32)

        for g in range(NCHUNK):
            b = g % NBUF
            pltpu.make_async_copy(
                x_hbm.at[pl.ds(base + g * CHUNK, CHUNK)], bufs[b], sems[b]
            ).wait()

            def acc_body(i, carry, b=b):
                vs = [
                    bufs[b][pl.ds((i * UNROLL + u) * L, L)]
                    for u in range(UNROLL)
                ]
                for v in vs:
                    plsc.addupdate_scatter(hist, [v], ones16)
                return carry

            lax.fori_loop(0, VPC // UNROLL, acc_body, 0)

            if g + NBUF < NCHUNK:
                start(g + NBUF, b)

        pltpu.sync_copy(hist, out_hbm.at[wid])

    return body(x)


def _tc_reduce_body(p_ref, o_ref):
    o_ref[...] = jnp.sum(p_ref[...], axis=0)


@jax.jit
def kernel(x):
    partials = _sc_bincount(x.astype(jnp.int32))
    return pl.pallas_call(
        _tc_reduce_body,
        grid=(16,),
        in_specs=[pl.BlockSpec((NW, NUM_BINS // 16), lambda i: (0, i))],
        out_specs=pl.BlockSpec((NUM_BINS // 16,), lambda i: (i,)),
        out_shape=jax.ShapeDtypeStruct((NUM_BINS,), jnp.int32),
    )(partials)


# speedup vs baseline: 1.0444x; 1.0444x over previous
import functools

import jax
import jax.numpy as jnp
from jax import lax
from jax.experimental import pallas as pl
from jax.experimental.pallas import tpu as pltpu
from jax.experimental.pallas import tpu_sc as plsc

NUM_BINS = 65536
N = 16777216
L = 16
NC = 2
NS = 16
NW = NC * NS
PER_W = N // NW
CHUNK = 16384
NCHUNK = PER_W // CHUNK
VPC = CHUNK // L
UNROLL = 16
NBUF = 3
ZUNROLL = 8


def _sc_bincount(x):
    mesh = plsc.VectorSubcoreMesh(core_axis_name="c", subcore_axis_name="s")

    @functools.partial(
        pl.kernel,
        mesh=mesh,
        compiler_params=pltpu.CompilerParams(needs_layout_passes=False),
        out_type=jax.ShapeDtypeStruct((NW, NUM_BINS), jnp.int32),
        scratch_types=[
            pltpu.VMEM((NUM_BINS,), jnp.int32),
        ]
        + [pltpu.VMEM((CHUNK,), jnp.int32)] * NBUF
        + [pltpu.SemaphoreType.DMA] * NBUF,
    )
    def body(x_hbm, out_hbm, hist, *rest):
        bufs = rest[:NBUF]
        sems = rest[NBUF:]
        c = lax.axis_index("c")
        s = lax.axis_index("s")
        wid = s * NC + c
        base = wid * PER_W

        def start(g, b):
            return pltpu.async_copy(
                x_hbm.at[pl.ds(base + g * CHUNK, CHUNK)], bufs[b], sems[b]
            )

        for b in range(NBUF):
            start(b, b)

        zeros16 = jnp.zeros((L,), jnp.int32)

        def zero_body(i, carry):
            for u in range(ZUNROLL):
                hist[pl.ds((i * ZUNROLL + u) * L, L)] = zeros16
            return carry

        lax.fori_loop(0, NUM_BINS // (L * ZUNROLL), zero_body, 0)

        ones16 = jnp.full((L,), 1, jnp.int32)

        for g in range(NCHUNK):
            b = g % NBUF
            pltpu.make_async_copy(
                x_hbm.at[pl.ds(base + g * CHUNK, CHUNK)], bufs[b], sems[b]
            ).wait()

            def acc_body(i, carry, b=b):
                vs = [
                    bufs[b][pl.ds((i * UNROLL + u) * L, L)]
                    for u in range(UNROLL)
                ]
                for v in vs:
                    plsc.addupdate_scatter(hist, [v], ones16)
                return carry

            lax.fori_loop(0, VPC // UNROLL, acc_body, 0)

            if g + NBUF < NCHUNK:
                start(g + NBUF, b)

        pltpu.sync_copy(hist, out_hbm.at[wid])

    return body(x)


def _tc_reduce_body(p_ref, o_ref):
    o_ref[...] = jnp.sum(p_ref[...], axis=0)


@jax.jit
def kernel(x):
    partials = _sc_bincount(x.astype(jnp.int32))
    return pl.pallas_call(
        _tc_reduce_body,
        grid=(8,),
        in_specs=[pl.BlockSpec((NW, NUM_BINS // 8), lambda i: (0, i))],
        out_specs=pl.BlockSpec((NUM_BINS // 8,), lambda i: (i,)),
        out_shape=jax.ShapeDtypeStruct((NUM_BINS,), jnp.int32),
    )(partials)


# speedup vs baseline: 1.0749x; 1.0292x over previous
import functools

import jax
import jax.numpy as jnp
from jax import lax
from jax.experimental import pallas as pl
from jax.experimental.pallas import tpu as pltpu
from jax.experimental.pallas import tpu_sc as plsc

NUM_BINS = 65536
N = 16777216
L = 16
NC = 2
NS = 16
NW = NC * NS
PER_W = N // NW
CHUNK = 16384
NCHUNK = PER_W // CHUNK
VPC = CHUNK // L
UNROLL = 16
NBUF = 3
ZUNROLL = 8


def _sc_bincount(x):
    mesh = plsc.VectorSubcoreMesh(core_axis_name="c", subcore_axis_name="s")

    @functools.partial(
        pl.kernel,
        mesh=mesh,
        compiler_params=pltpu.CompilerParams(needs_layout_passes=False),
        out_type=jax.ShapeDtypeStruct((NW, NUM_BINS), jnp.int32),
        scratch_types=[
            pltpu.VMEM((NUM_BINS,), jnp.int32),
        ]
        + [pltpu.VMEM((CHUNK,), jnp.int32)] * NBUF
        + [pltpu.SemaphoreType.DMA] * NBUF,
    )
    def body(x_hbm, out_hbm, hist, *rest):
        bufs = rest[:NBUF]
        sems = rest[NBUF:]
        c = lax.axis_index("c")
        s = lax.axis_index("s")
        wid = s * NC + c
        base = wid * PER_W

        def start(g, b):
            return pltpu.async_copy(
                x_hbm.at[pl.ds(base + g * CHUNK, CHUNK)], bufs[b], sems[b]
            )

        for b in range(NBUF):
            start(b, b)

        zeros16 = jnp.zeros((L,), jnp.int32)

        def zero_body(i, carry):
            for u in range(ZUNROLL):
                hist[pl.ds((i * ZUNROLL + u) * L, L)] = zeros16
            return carry

        lax.fori_loop(0, NUM_BINS // (L * ZUNROLL), zero_body, 0)

        ones16 = jnp.full((L,), 1, jnp.int32)

        for g in range(NCHUNK):
            b = g % NBUF
            pltpu.make_async_copy(
                x_hbm.at[pl.ds(base + g * CHUNK, CHUNK)], bufs[b], sems[b]
            ).wait()

            def acc_body(i, carry, b=b):
                vs = [
                    bufs[b][pl.ds((i * UNROLL + u) * L, L)]
                    for u in range(UNROLL)
                ]
                for v in vs:
                    plsc.addupdate_scatter(hist, [v], ones16)
                return carry

            lax.fori_loop(0, VPC // UNROLL, acc_body, 0)

            if g + NBUF < NCHUNK:
                start(g + NBUF, b)

        pltpu.sync_copy(hist, out_hbm.at[wid])

    return body(x)


def _tc_reduce_body(p_ref, o_ref):
    o_ref[...] = jnp.sum(p_ref[...], axis=0)


@jax.jit
def kernel(x):
    partials = _sc_bincount(x.astype(jnp.int32))
    return pl.pallas_call(
        _tc_reduce_body,
        out_shape=jax.ShapeDtypeStruct((NUM_BINS,), jnp.int32),
    )(partials)
